# Initial kernel scaffold; baseline (speedup 1.0000x reference)
import jax
import jax.numpy as jnp
from jax.experimental import pallas as pl


def kernel(x, pe_weight):
    raise NotImplementedError("write your pallas kernel here")


# speedup vs baseline: 1.6803x; 1.6803x over previous
import jax
import jax.numpy as jnp
from jax.experimental import pallas as pl


_BS = 512


def _add_kernel(x_ref, pe_ref, o_ref):
    o_ref[...] = x_ref[...] + pe_ref[...]


def kernel(x, pe_weight):
    batch, seq_len, d_model = x.shape
    ns = seq_len // _BS
    grid = (ns, batch)
    return pl.pallas_call(
        _add_kernel,
        grid=grid,
        in_specs=[
            pl.BlockSpec((1, _BS, d_model), lambda i, b: (b, i, 0)),
            pl.BlockSpec((_BS, d_model), lambda i, b: (i, 0)),
        ],
        out_specs=pl.BlockSpec((1, _BS, d_model), lambda i, b: (b, i, 0)),
        out_shape=jax.ShapeDtypeStruct((batch, seq_len, d_model), x.dtype),
    )(x, pe_weight)


# speedup vs baseline: 1.8846x; 1.1216x over previous
import jax
import jax.numpy as jnp
from jax.experimental import pallas as pl


_BS = 1024


def _add_kernel(x_ref, pe_ref, o_ref):
    o_ref[...] = x_ref[...] + pe_ref[...]


def kernel(x, pe_weight):
    batch, seq_len, d_model = x.shape
    ns = seq_len // _BS
    grid = (ns, batch)
    return pl.pallas_call(
        _add_kernel,
        grid=grid,
        in_specs=[
            pl.BlockSpec((1, _BS, d_model), lambda i, b: (b, i, 0)),
            pl.BlockSpec((_BS, d_model), lambda i, b: (i, 0)),
        ],
        out_specs=pl.BlockSpec((1, _BS, d_model), lambda i, b: (b, i, 0)),
        out_shape=jax.ShapeDtypeStruct((batch, seq_len, d_model), x.dtype),
    )(x, pe_weight)
